# baseline (speedup 1.0000x reference)
import jax
import jax.numpy as jnp
from jax import lax
from jax.experimental import pallas as pl
from jax.experimental.pallas import tpu as pltpu
from jax.experimental.pallas import tpu_sc as plsc

_SCALE = 64.0
_THRESH = 0.5

_B = 1024
_V = 100000
_L = 16
_NC, _NS = 2, 16
_NW = _NC * _NS
_CPW = _B // _NW

_VBLK = 2000


def _dense_body(x_ref, o_ref):
    x = x_ref[...]
    o_ref[...] = jnp.where(x > _THRESH, jnp.float32(0.0), x * _SCALE)


def _dense_t(logits_t):
    return pl.pallas_call(
        _dense_body,
        out_shape=jax.ShapeDtypeStruct((_V, _B), jnp.float32),
        grid=(_V // _VBLK,),
        in_specs=[pl.BlockSpec((_VBLK, _B), lambda i: (i, 0))],
        out_specs=pl.BlockSpec((_VBLK, _B), lambda i: (i, 0)),
    )(logits_t)


_K = 16


def _sc_fix_body(logits_hbm, labels_hbm, out_hbm, lab_v, tiles_v,
                 sem_r, sem_w):
    wid = lax.axis_index("s") * _NC + lax.axis_index("c")
    cb = pl.multiple_of((wid >> 2) * 128, 128)
    p0 = (wid & 3) * _CPW
    pltpu.sync_copy(labels_hbm.at[pl.ds(cb, 128)], lab_v)
    lane = lax.iota(jnp.int32, _L)

    def vbase_at(p):
        lv = lab_v[pl.ds(pl.multiple_of((p >> 4) << 4, _L), _L)]
        lab_p = jnp.sum(jnp.where(lane == (p & 15), lv, 0))
        return pl.multiple_of(lab_p & -8, 8), lab_p >> 3

    def batch(bi, carry):
        jb = p0 + bi * _K
        reads = []
        for b in range(_K):
            vb, _ = vbase_at(jb + b)
            reads.append(pltpu.async_copy(
                logits_hbm.at[pl.ds(vb, 8), pl.ds(cb, 128)],
                tiles_v.at[b], sem_r))
        for h in reads:
            h.wait()
        writes = []
        for b in range(_K):
            vb, hi = vbase_at(jb + b)
            for v in range(8):
                labv = lab_v[pl.ds(v * _L, _L)]
                rowmatch = (labv >> 3) == hi
                labmod = labv & 7
                for r in range(8):
                    x = tiles_v[b, r, pl.ds(v * _L, _L)]
                    ex = rowmatch & (labmod == r)
                    keep = (x <= _THRESH) | ex
                    tiles_v[b, r, pl.ds(v * _L, _L)] = jnp.where(
                        keep, x * _SCALE, jnp.float32(0.0))
            writes.append(pltpu.async_copy(
                tiles_v.at[b], out_hbm.at[pl.ds(vb, 8), pl.ds(cb, 128)],
                sem_w))
        for h in writes:
            h.wait()
        return carry

    lax.fori_loop(0, _CPW // _K, batch, 0)


def _sc_fix(logits_t, labels, out_ref):
    mesh = plsc.VectorSubcoreMesh(
        core_axis_name="c", subcore_axis_name="s",
        num_cores=_NC, num_subcores=_NS)
    k = pl.kernel(
        _sc_fix_body,
        out_type=(),
        mesh=mesh,
        compiler_params=pltpu.CompilerParams(needs_layout_passes=False),
        scratch_types=[
            pltpu.VMEM((128,), jnp.int32),
            pltpu.VMEM((_K, 8, 128), jnp.float32),
            pltpu.SemaphoreType.DMA,
            pltpu.SemaphoreType.DMA,
        ],
    )
    k(logits_t, labels, out_ref)


def kernel(logits, labels):
    logits_t = logits.T
    dense_t = _dense_t(logits_t)
    out_ref = jax.new_ref(dense_t)
    _sc_fix(logits_t, labels, out_ref)
    return out_ref[...].T


# speedup vs baseline: 1.0089x; 1.0089x over previous
import jax
import jax.numpy as jnp
from jax import lax
from jax.experimental import pallas as pl
from jax.experimental.pallas import tpu as pltpu
from jax.experimental.pallas import tpu_sc as plsc

_SCALE = 64.0
_THRESH = 0.5

_B = 1024
_V = 100000
_L = 16
_NC, _NS = 2, 16
_NW = _NC * _NS
_CPW = _B // _NW

_VBLK = 4000


def _dense_body(x_ref, o_ref):
    x = x_ref[...]
    o_ref[...] = jnp.where(x > _THRESH, jnp.float32(0.0), x * _SCALE)


def _dense_t(logits_t):
    return pl.pallas_call(
        _dense_body,
        out_shape=jax.ShapeDtypeStruct((_V, _B), jnp.float32),
        grid=(_V // _VBLK,),
        in_specs=[pl.BlockSpec((_VBLK, _B), lambda i: (i, 0))],
        out_specs=pl.BlockSpec((_VBLK, _B), lambda i: (i, 0)),
        compiler_params=pltpu.CompilerParams(
            vmem_limit_bytes=100 * 1024 * 1024),
    )(logits_t)


_K = 8


def _sc_fix_body(logits_hbm, labels_hbm, out_hbm, lab_v, tiles_v,
                 sem_r, sem_w):
    wid = lax.axis_index("s") * _NC + lax.axis_index("c")
    cb = pl.multiple_of((wid >> 2) * 128, 128)
    p0 = (wid & 3) * _CPW
    pltpu.sync_copy(labels_hbm.at[pl.ds(cb, 128)], lab_v)
    lane = lax.iota(jnp.int32, _L)

    def vbase_at(p):
        lv = lab_v[pl.ds(pl.multiple_of((p >> 4) << 4, _L), _L)]
        lab_p = jnp.sum(jnp.where(lane == (p & 15), lv, 0))
        return pl.multiple_of(lab_p & -8, 8), lab_p >> 3

    def batch(bi, carry):
        jb = p0 + bi * _K
        reads = []
        for b in range(_K):
            vb, _ = vbase_at(jb + b)
            reads.append(pltpu.async_copy(
                logits_hbm.at[pl.ds(vb, 8), pl.ds(cb, 128)],
                tiles_v.at[b], sem_r))
        for h in reads:
            h.wait()
        writes = []
        for b in range(_K):
            vb, hi = vbase_at(jb + b)
            for v in range(8):
                labv = lab_v[pl.ds(v * _L, _L)]
                rowmatch = (labv >> 3) == hi
                labmod = labv & 7
                for r in range(8):
                    x = tiles_v[b, r, pl.ds(v * _L, _L)]
                    ex = rowmatch & (labmod == r)
                    keep = (x <= _THRESH) | ex
                    tiles_v[b, r, pl.ds(v * _L, _L)] = jnp.where(
                        keep, x * _SCALE, jnp.float32(0.0))
            writes.append(pltpu.async_copy(
                tiles_v.at[b], out_hbm.at[pl.ds(vb, 8), pl.ds(cb, 128)],
                sem_w))
        for h in writes:
            h.wait()
        return carry

    lax.fori_loop(0, _CPW // _K, batch, 0)


def _sc_fix(logits_t, labels, out_ref):
    mesh = plsc.VectorSubcoreMesh(
        core_axis_name="c", subcore_axis_name="s",
        num_cores=_NC, num_subcores=_NS)
    k = pl.kernel(
        _sc_fix_body,
        out_type=(),
        mesh=mesh,
        compiler_params=pltpu.CompilerParams(needs_layout_passes=False),
        scratch_types=[
            pltpu.VMEM((128,), jnp.int32),
            pltpu.VMEM((_K, 8, 128), jnp.float32),
            pltpu.SemaphoreType.DMA,
            pltpu.SemaphoreType.DMA,
        ],
    )
    k(logits_t, labels, out_ref)


def kernel(logits, labels):
    logits_t = logits.T
    dense_t = _dense_t(logits_t)
    out_ref = jax.new_ref(dense_t)
    _sc_fix(logits_t, labels, out_ref)
    return out_ref[...].T
